# baseline (speedup 1.0000x reference)
import functools

import jax
import jax.numpy as jnp
from jax import lax
from jax.experimental import pallas as pl
from jax.experimental.pallas import tpu as pltpu
from jax.experimental.pallas import tpu_sc as plsc

_INPUT_DIM = 4096
_NUM_WORKERS = 32


def kernel(raw_data, emb_value, emb_alpha, factor):
    B, C, D, H, W = raw_data.shape
    n_planes = B * C * D
    steps = n_planes // _NUM_WORKERS
    col_chunks = W // 16

    raw_planes = raw_data.reshape(n_planes, H, W)
    vtab_host = jnp.clip(emb_value.reshape(-1), 0.0, 1.0)
    atab_host = jnp.clip(emb_alpha.reshape(-1), 0.0, 1.0) * jnp.float32(factor)

    mesh = plsc.VectorSubcoreMesh(core_axis_name="c", subcore_axis_name="s")

    @functools.partial(
        pl.kernel,
        mesh=mesh,
        compiler_params=pltpu.CompilerParams(needs_layout_passes=False),
        out_type=jax.ShapeDtypeStruct((B, 4, D, H, W), jnp.float32),
        scratch_types=[
            pltpu.VMEM((_INPUT_DIM,), jnp.float32),
            pltpu.VMEM((_INPUT_DIM,), jnp.float32),
            pltpu.VMEM((2, H, W), jnp.float32),
            pltpu.VMEM((2, H, W), jnp.float32),
            pltpu.VMEM((2, H, W), jnp.float32),
            pltpu.SemaphoreType.DMA((2,)),
            pltpu.SemaphoreType.DMA((2,)),
        ],
    )
    def _lut_kernel(raw_hbm, vtab_hbm, atab_hbm, out_hbm, vtab, atab, rawb,
                    vbuf, abuf, in_sem, out_sem):
        wid = lax.axis_index("s") * 2 + lax.axis_index("c")
        pltpu.sync_copy(vtab_hbm, vtab)
        pltpu.sync_copy(atab_hbm, atab)
        base = wid * steps

        def start_in(g, slot):
            return pltpu.async_copy(
                raw_hbm.at[base + g], rawb.at[slot], in_sem.at[slot]
            )

        def start_out(g, slot):
            p = base + g
            b = p // D
            dpl = p - b * D
            return [
                pltpu.async_copy(
                    vbuf.at[slot], out_hbm.at[b, c, dpl], out_sem.at[slot]
                )
                for c in range(3)
            ] + [
                pltpu.async_copy(
                    abuf.at[slot], out_hbm.at[b, 3, dpl], out_sem.at[slot]
                )
            ]

        in_handles = [None, None]
        out_handles = [None, None]
        in_handles[0] = start_in(0, 0)
        if steps > 1:
            in_handles[1] = start_in(1, 1)

        for g in range(steps):
            slot = g % 2
            in_handles[slot].wait()
            if out_handles[slot] is not None:
                for h in out_handles[slot]:
                    h.wait()

            @plsc.parallel_loop(0, H, 1, unroll=4)
            def inner(r, slot=slot):
                for cc in range(col_chunks):
                    x = rawb[slot, r, pl.ds(cc * 16, 16)]
                    idx = (x * (_INPUT_DIM - 1)).astype(jnp.int32)
                    vbuf[slot, r, pl.ds(cc * 16, 16)] = plsc.load_gather(
                        vtab, [idx]
                    )
                    abuf[slot, r, pl.ds(cc * 16, 16)] = plsc.load_gather(
                        atab, [idx]
                    )

            if g + 2 < steps:
                in_handles[slot] = start_in(g + 2, slot)
            out_handles[slot] = start_out(g, slot)

        for hs in out_handles:
            if hs is not None:
                for h in hs:
                    h.wait()

    return _lut_kernel(raw_planes, vtab_host, atab_host)


# speedup vs baseline: 1.0049x; 1.0049x over previous
import functools

import jax
import jax.numpy as jnp
from jax import lax
from jax.experimental import pallas as pl
from jax.experimental.pallas import tpu as pltpu
from jax.experimental.pallas import tpu_sc as plsc

_INPUT_DIM = 4096
_NUM_WORKERS = 32


def kernel(raw_data, emb_value, emb_alpha, factor):
    B, C, D, H, W = raw_data.shape
    n_planes = B * C * D
    steps = n_planes // _NUM_WORKERS
    col_chunks = W // 16

    raw_planes = raw_data.reshape(n_planes, H, W)
    fsplat = jnp.full((16,), factor, dtype=jnp.float32)

    mesh = plsc.VectorSubcoreMesh(core_axis_name="c", subcore_axis_name="s")

    @functools.partial(
        pl.kernel,
        mesh=mesh,
        compiler_params=pltpu.CompilerParams(needs_layout_passes=False),
        out_type=jax.ShapeDtypeStruct((B, 4, D, H, W), jnp.float32),
        scratch_types=[
            pltpu.VMEM((_INPUT_DIM,), jnp.float32),
            pltpu.VMEM((_INPUT_DIM,), jnp.float32),
            pltpu.VMEM((16,), jnp.float32),
            pltpu.VMEM((2, H, W), jnp.float32),
            pltpu.VMEM((2, H, W), jnp.float32),
            pltpu.VMEM((2, H, W), jnp.float32),
            pltpu.SemaphoreType.DMA((2,)),
            pltpu.SemaphoreType.DMA((2,)),
        ],
    )
    def _lut_kernel(raw_hbm, vtab_hbm, atab_hbm, f_hbm, out_hbm, vtab, atab,
                    fbuf, rawb, vbuf, abuf, in_sem, out_sem):
        wid = lax.axis_index("s") * 2 + lax.axis_index("c")
        base = wid * steps

        def start_in(g, slot):
            return pltpu.async_copy(
                raw_hbm.at[base + g], rawb.at[slot], in_sem.at[slot]
            )

        def start_out(g, slot):
            p = base + g
            b = p // D
            dpl = p - b * D
            return [
                pltpu.async_copy(
                    vbuf.at[slot], out_hbm.at[b, c, dpl], out_sem.at[slot]
                )
                for c in range(3)
            ] + [
                pltpu.async_copy(
                    abuf.at[slot], out_hbm.at[b, 3, dpl], out_sem.at[slot]
                )
            ]

        in_handles = [None, None]
        out_handles = [None, None]
        in_handles[0] = start_in(0, 0)
        if steps > 1:
            in_handles[1] = start_in(1, 1)

        pltpu.sync_copy(vtab_hbm, vtab)
        pltpu.sync_copy(atab_hbm, atab)
        pltpu.sync_copy(f_hbm, fbuf)
        fvec = fbuf[...]

        @plsc.parallel_loop(0, _INPUT_DIM, 16, unroll=4)
        def _prep(t):
            vtab[pl.ds(t, 16)] = jnp.clip(vtab[pl.ds(t, 16)], 0.0, 1.0)
            atab[pl.ds(t, 16)] = (
                jnp.clip(atab[pl.ds(t, 16)], 0.0, 1.0) * fvec
            )

        for g in range(steps):
            slot = g % 2
            in_handles[slot].wait()
            if out_handles[slot] is not None:
                for h in out_handles[slot]:
                    h.wait()

            @plsc.parallel_loop(0, H, 1, unroll=2)
            def inner(r, slot=slot):
                for cc in range(col_chunks):
                    x = rawb[slot, r, pl.ds(cc * 16, 16)]
                    idx = (x * (_INPUT_DIM - 1)).astype(jnp.int32)
                    vbuf[slot, r, pl.ds(cc * 16, 16)] = plsc.load_gather(
                        vtab, [idx]
                    )
                    abuf[slot, r, pl.ds(cc * 16, 16)] = plsc.load_gather(
                        atab, [idx]
                    )

            if g + 2 < steps:
                in_handles[slot] = start_in(g + 2, slot)
            out_handles[slot] = start_out(g, slot)

        for hs in out_handles:
            if hs is not None:
                for h in hs:
                    h.wait()

    return _lut_kernel(
        raw_planes, emb_value.reshape(-1), emb_alpha.reshape(-1), fsplat
    )
